# baseline (speedup 1.0000x reference)
import functools

import jax
import jax.numpy as jnp
from jax import lax
from jax.experimental import pallas as pl
from jax.experimental.pallas import tpu as pltpu
from jax.experimental.pallas import tpu_sc as plsc

N = 10000
E = 320000
D = 128
NP = 10240

NC = 2
NS = 16
NW = NC * NS
CHUNK = 128
NCHUNK = 80
EPW = NCHUNK * CHUNK
EP = EPW * NW
ROWS_PER_TILE = NP // NS

_mesh = plsc.VectorSubcoreMesh(core_axis_name="c", subcore_axis_name="s")


def _make_agg(with_deg):
    out_type = [jax.ShapeDtypeStruct((NC, NP, D), jnp.float32)]
    scratch = [
        pltpu.VMEM_SHARED((NP, D), jnp.float32),
        pltpu.VMEM((CHUNK,), jnp.int32),
        pltpu.VMEM((CHUNK,), jnp.int32),
        pltpu.VMEM((CHUNK,), jnp.int32),
        pltpu.VMEM((CHUNK,), jnp.int32),
        pltpu.VMEM((CHUNK, D), jnp.float32),
        pltpu.VMEM((CHUNK, D), jnp.float32),
        pltpu.SemaphoreType.DMA,
        pltpu.SemaphoreType.DMA,
        pltpu.SemaphoreType.DMA,
        pltpu.SemaphoreType.DMA,
        pltpu.SemaphoreType.DMA,
        pltpu.SemaphoreType.DMA,
        pltpu.SemaphoreType.DMA,
        pltpu.SemaphoreType.DMA,
    ]
    if with_deg:
        out_type += [jax.ShapeDtypeStruct((NP,), jnp.float32),
                     jax.ShapeDtypeStruct((NP,), jnp.float32)]
        scratch += [
            pltpu.VMEM_SHARED((NP,), jnp.float32),
            pltpu.VMEM((CHUNK,), jnp.float32),
            pltpu.SemaphoreType.DMA,
            pltpu.SemaphoreType.DMA,
        ]

    def body(*refs):
        if with_deg:
            (table, src3, dst3, zrows, zdeg, out_agg, out_deg0, out_deg1,
             acc, sb0, sb1, db0, db1, rows0, rows1,
             si0, si1, di0, di1, gg0, gg1, gs0, gs1,
             dega, ones_v, gd0, gd1) = refs
        else:
            (table, src3, dst3, zrows, out_agg,
             acc, sb0, sb1, db0, db1, rows0, rows1,
             si0, si1, di0, di1, gg0, gg1, gs0, gs1) = refs
            dega = ones_v = gd0 = gd1 = None
        sb = (sb0, sb1)
        db = (db0, db1)
        rows = (rows0, rows1)
        si = (si0, si1)
        di = (di0, di1)
        gg = (gg0, gg1)
        gs = (gs0, gs1)
        gd = (gd0, gd1)
        c = lax.axis_index("c")
        s = lax.axis_index("s")
        wid = c * NS + s
        r0 = s * ROWS_PER_TILE

        pltpu.sync_copy(zrows.at[pl.ds(r0, ROWS_PER_TILE)],
                        acc.at[pl.ds(r0, ROWS_PER_TILE)])
        if with_deg:
            @pl.when(s == 0)
            def _():
                pltpu.sync_copy(zdeg, dega)
            for j in range(CHUNK // 16):
                ones_v[pl.ds(j * 16, 16)] = jnp.full((16,), 1.0, jnp.float32)
        plsc.subcore_barrier()

        def fetch(j, b):
            jc = jnp.minimum(j, NCHUNK - 1)
            pltpu.async_copy(src3.at[wid, jc], sb[b], si[b])
            pltpu.async_copy(dst3.at[wid, jc], db[b], di[b])

        def wait_idx(j, b):
            jc = jnp.minimum(j, NCHUNK - 1)
            pltpu.make_async_copy(src3.at[wid, jc], sb[b], si[b]).wait()
            pltpu.make_async_copy(dst3.at[wid, jc], db[b], di[b]).wait()

        def step(j, b, last):
            o = 1 - b
            pltpu.make_async_copy(table.at[sb[b]], rows[b], gg[b]).wait()
            pltpu.async_copy(rows[b], acc.at[db[b]], gs[b], add=True)
            if with_deg:
                pltpu.async_copy(ones_v, dega.at[db[b]], gd[b], add=True)
            if not last:
                wait_idx(j + 1, o)
                pltpu.async_copy(table.at[sb[o]], rows[o], gg[o])
            pltpu.make_async_copy(rows[b], acc.at[db[b]], gs[b]).wait()
            if with_deg:
                pltpu.make_async_copy(ones_v, dega.at[db[b]], gd[b]).wait()
            if not last:
                fetch(j + 2, b)

        fetch(0, 0)
        fetch(1, 1)
        wait_idx(0, 0)
        pltpu.async_copy(table.at[sb[0]], rows0, gg0)
        step(0, 0, last=False)

        def pair(i, carry):
            step(2 * i + 1, 1, last=False)
            step(2 * i + 2, 0, last=False)
            return carry

        lax.fori_loop(0, (NCHUNK - 2) // 2, pair, 0)
        step(NCHUNK - 1, 1, last=True)
        wait_idx(NCHUNK, 0)
        plsc.subcore_barrier()

        pltpu.sync_copy(acc.at[pl.ds(r0, ROWS_PER_TILE)],
                        out_agg.at[c, pl.ds(r0, ROWS_PER_TILE)])
        if with_deg:
            @pl.when(c == 0)
            def _():
                pltpu.sync_copy(dega.at[pl.ds(r0, ROWS_PER_TILE)],
                                out_deg0.at[pl.ds(r0, ROWS_PER_TILE)])

            @pl.when(c == 1)
            def _():
                pltpu.sync_copy(dega.at[pl.ds(r0, ROWS_PER_TILE)],
                                out_deg1.at[pl.ds(r0, ROWS_PER_TILE)])

    return pl.kernel(body, mesh=_mesh, out_type=tuple(out_type),
                     scratch_types=scratch)


_agg_deg = _make_agg(with_deg=True)
_agg = _make_agg(with_deg=False)

_R = 1024
_RS = _R // D


def _dense_body(p_ref, d0_ref, d1_ref, h_ref, wl_ref, bl_ref, wr_ref, o_ref,
                *, selu):
    agg = p_ref[0] + p_ref[1]
    deg = d0_ref[...] + d1_ref[...]
    r = 1.0 / jnp.maximum(deg, 1.0)
    a3 = agg.reshape(_RS, D, D) * r[:, :, None]
    a = a3.reshape(_R, D)
    out = (lax.dot_general(a, wl_ref[...], (((1,), (1,)), ((), ())),
                           preferred_element_type=jnp.float32)
           + bl_ref[...]
           + lax.dot_general(h_ref[...], wr_ref[...], (((1,), (1,)), ((), ())),
                             preferred_element_type=jnp.float32))
    if selu:
        alpha = 1.6732632423543772
        scale = 1.0507009873554805
        out = scale * jnp.where(out > 0, out, alpha * (jnp.exp(out) - 1.0))
    o_ref[...] = out


def _dense(p, d0, d1, h, Wl, bl2, Wr, selu):
    return pl.pallas_call(
        functools.partial(_dense_body, selu=selu),
        grid=(NP // _R,),
        in_specs=[
            pl.BlockSpec((NC, _R, D), lambda i: (0, i, 0)),
            pl.BlockSpec((_RS, D), lambda i: (i, 0)),
            pl.BlockSpec((_RS, D), lambda i: (i, 0)),
            pl.BlockSpec((_R, D), lambda i: (i, 0)),
            pl.BlockSpec((D, D), lambda i: (0, 0)),
            pl.BlockSpec((1, D), lambda i: (0, 0)),
            pl.BlockSpec((D, D), lambda i: (0, 0)),
        ],
        out_specs=pl.BlockSpec((_R, D), lambda i: (i, 0)),
        out_shape=jax.ShapeDtypeStruct((NP, D), jnp.float32),
    )(p, d0, d1, h, Wl, bl2, Wr)


def kernel(x, adj_t, W1l, b1l, W1r, W2l, b2l, W2r):
    pad = jnp.full((EP - E,), N, jnp.int32)
    src = jnp.concatenate([adj_t[0], pad]).reshape(NW, NCHUNK, CHUNK)
    dst = jnp.concatenate([adj_t[1], pad]).reshape(NW, NCHUNK, CHUNK)
    xp = jnp.pad(x, ((0, NP - N), (0, 0)))
    zrows = jnp.zeros((NP, D), jnp.float32)
    zdeg = jnp.zeros((NP,), jnp.float32)
    p1, deg0, deg1 = _agg_deg(xp, src, dst, zrows, zdeg)
    d0 = deg0.reshape(NP // D, D)
    d1 = deg1.reshape(NP // D, D)
    h1 = _dense(p1, d0, d1, xp, W1l, b1l.reshape(1, D), W1r, selu=True)
    p2, = _agg(h1, src, dst, zrows)
    out = _dense(p2, d0, d1, h1, W2l, b2l.reshape(1, D), W2r, selu=False)
    return out[:N]


# speedup vs baseline: 1.0496x; 1.0496x over previous
import functools

import jax
import jax.numpy as jnp
from jax import lax
from jax.experimental import pallas as pl
from jax.experimental.pallas import tpu as pltpu
from jax.experimental.pallas import tpu_sc as plsc

N = 10000
E = 320000
D = 128
NP = 10240

NC = 2
NS = 16
NW = NC * NS
SLABW = 128
NROW = 80
CHUNK = 64
NCHUNK = 160
EPW = NROW * SLABW
EP = EPW * NW
ROWS_PER_TILE = NP // NS

_mesh = plsc.VectorSubcoreMesh(core_axis_name="c", subcore_axis_name="s")


def _make_agg(with_deg):
    out_type = [jax.ShapeDtypeStruct((NC, NP, D), jnp.float32)]
    scratch = [
        pltpu.VMEM_SHARED((NP, D), jnp.float32),
        pltpu.VMEM((NROW, SLABW), jnp.int32),
        pltpu.VMEM((NROW, SLABW), jnp.int32),
        pltpu.VMEM((CHUNK, D), jnp.float32),
        pltpu.VMEM((CHUNK, D), jnp.float32),
        pltpu.VMEM((CHUNK, D), jnp.float32),
        pltpu.SemaphoreType.DMA,
        pltpu.SemaphoreType.DMA,
        pltpu.SemaphoreType.DMA,
        pltpu.SemaphoreType.DMA,
        pltpu.SemaphoreType.DMA,
        pltpu.SemaphoreType.DMA,
    ]
    if with_deg:
        out_type += [jax.ShapeDtypeStruct((NP,), jnp.float32),
                     jax.ShapeDtypeStruct((NP,), jnp.float32)]
        scratch += [
            pltpu.VMEM_SHARED((NP,), jnp.float32),
            pltpu.VMEM((CHUNK,), jnp.float32),
            pltpu.SemaphoreType.DMA,
            pltpu.SemaphoreType.DMA,
            pltpu.SemaphoreType.DMA,
        ]

    def body(*refs):
        if with_deg:
            (table, src3, dst3, zrows, zdeg, out_agg, out_deg0, out_deg1,
             acc, srcw, dstw, rows0, rows1, rows2,
             gg0, gg1, gg2, gs0, gs1, gs2,
             dega, ones_v, gd0, gd1, gd2) = refs
        else:
            (table, src3, dst3, zrows, out_agg,
             acc, srcw, dstw, rows0, rows1, rows2,
             gg0, gg1, gg2, gs0, gs1, gs2) = refs
            dega = ones_v = None
            gd0 = gd1 = gd2 = None
        rows = (rows0, rows1, rows2)
        gg = (gg0, gg1, gg2)
        gs = (gs0, gs1, gs2)
        gd = (gd0, gd1, gd2)
        c = lax.axis_index("c")
        s = lax.axis_index("s")
        wid = c * NS + s
        r0 = s * ROWS_PER_TILE

        pltpu.sync_copy(src3.at[wid], srcw)
        pltpu.sync_copy(dst3.at[wid], dstw)
        pltpu.sync_copy(zrows.at[pl.ds(r0, ROWS_PER_TILE)],
                        acc.at[pl.ds(r0, ROWS_PER_TILE)])
        if with_deg:
            @pl.when(s == 0)
            def _():
                pltpu.sync_copy(zdeg, dega)
            for j in range(CHUNK // 16):
                ones_v[pl.ds(j * 16, 16)] = jnp.full((16,), 1.0, jnp.float32)
        plsc.subcore_barrier()

        def sidx(w, j):
            jc = jnp.minimum(j, NCHUNK - 1)
            return w.at[jc // 2, pl.ds((jc % 2) * CHUNK, CHUNK)]

        def step(j, b, last):
            pltpu.make_async_copy(table.at[sidx(srcw, j)], rows[b], gg[b]).wait()
            pltpu.async_copy(rows[b], acc.at[sidx(dstw, j)], gs[b], add=True)
            if with_deg:
                pltpu.async_copy(ones_v, dega.at[sidx(dstw, j)], gd[b],
                                 add=True)
            p = (b + 2) % 3
            if not last:
                pltpu.make_async_copy(rows[p], acc.at[sidx(dstw, j - 1)],
                                      gs[p]).wait()
                if with_deg:
                    pltpu.make_async_copy(ones_v, dega.at[sidx(dstw, j - 1)],
                                          gd[p]).wait()
                pltpu.async_copy(table.at[sidx(srcw, j + 2)], rows[p], gg[p])

        pltpu.async_copy(table.at[sidx(srcw, 0)], rows0, gg0)
        pltpu.async_copy(table.at[sidx(srcw, 1)], rows1, gg1)
        pltpu.make_async_copy(table.at[sidx(srcw, 0)], rows0, gg0).wait()
        pltpu.async_copy(rows0, acc.at[sidx(dstw, 0)], gs0, add=True)
        if with_deg:
            pltpu.async_copy(ones_v, dega.at[sidx(dstw, 0)], gd0, add=True)
        pltpu.async_copy(table.at[sidx(srcw, 2)], rows2, gg2)

        def triple(i, carry):
            step(3 * i + 1, 1, last=False)
            step(3 * i + 2, 2, last=False)
            step(3 * i + 3, 0, last=False)
            return carry

        lax.fori_loop(0, (NCHUNK - 1) // 3, triple, 0)
        bl = (NCHUNK - 1) % 3
        pltpu.make_async_copy(rows[bl], acc.at[sidx(dstw, NCHUNK - 1)],
                              gs[bl]).wait()
        if with_deg:
            pltpu.make_async_copy(ones_v, dega.at[sidx(dstw, NCHUNK - 1)],
                                  gd[bl]).wait()
        pltpu.make_async_copy(table.at[sidx(srcw, NCHUNK)],
                              rows[(bl + 1) % 3], gg[(bl + 1) % 3]).wait()
        pltpu.make_async_copy(table.at[sidx(srcw, NCHUNK + 1)],
                              rows[(bl + 2) % 3], gg[(bl + 2) % 3]).wait()
        plsc.subcore_barrier()

        pltpu.sync_copy(acc.at[pl.ds(r0, ROWS_PER_TILE)],
                        out_agg.at[c, pl.ds(r0, ROWS_PER_TILE)])
        if with_deg:
            @pl.when(c == 0)
            def _():
                pltpu.sync_copy(dega.at[pl.ds(r0, ROWS_PER_TILE)],
                                out_deg0.at[pl.ds(r0, ROWS_PER_TILE)])

            @pl.when(c == 1)
            def _():
                pltpu.sync_copy(dega.at[pl.ds(r0, ROWS_PER_TILE)],
                                out_deg1.at[pl.ds(r0, ROWS_PER_TILE)])

    return pl.kernel(body, mesh=_mesh, out_type=tuple(out_type),
                     scratch_types=scratch)


_agg_deg = _make_agg(with_deg=True)
_agg = _make_agg(with_deg=False)

_R = 1024
_RS = _R // D


def _dense_body(p_ref, d0_ref, d1_ref, h_ref, wl_ref, bl_ref, wr_ref, o_ref,
                *, selu):
    agg = p_ref[0] + p_ref[1]
    deg = d0_ref[...] + d1_ref[...]
    r = 1.0 / jnp.maximum(deg, 1.0)
    a3 = agg.reshape(_RS, D, D) * r[:, :, None]
    a = a3.reshape(_R, D)
    out = (lax.dot_general(a, wl_ref[...], (((1,), (1,)), ((), ())),
                           preferred_element_type=jnp.float32)
           + bl_ref[...]
           + lax.dot_general(h_ref[...], wr_ref[...], (((1,), (1,)), ((), ())),
                             preferred_element_type=jnp.float32))
    if selu:
        alpha = 1.6732632423543772
        scale = 1.0507009873554805
        out = scale * jnp.where(out > 0, out, alpha * (jnp.exp(out) - 1.0))
    o_ref[...] = out


def _dense(p, d0, d1, h, Wl, bl2, Wr, selu):
    return pl.pallas_call(
        functools.partial(_dense_body, selu=selu),
        grid=(NP // _R,),
        in_specs=[
            pl.BlockSpec((NC, _R, D), lambda i: (0, i, 0)),
            pl.BlockSpec((_RS, D), lambda i: (i, 0)),
            pl.BlockSpec((_RS, D), lambda i: (i, 0)),
            pl.BlockSpec((_R, D), lambda i: (i, 0)),
            pl.BlockSpec((D, D), lambda i: (0, 0)),
            pl.BlockSpec((1, D), lambda i: (0, 0)),
            pl.BlockSpec((D, D), lambda i: (0, 0)),
        ],
        out_specs=pl.BlockSpec((_R, D), lambda i: (i, 0)),
        out_shape=jax.ShapeDtypeStruct((NP, D), jnp.float32),
    )(p, d0, d1, h, Wl, bl2, Wr)


def kernel(x, adj_t, W1l, b1l, W1r, W2l, b2l, W2r):
    pad = jnp.full((EP - E,), N, jnp.int32)
    src = jnp.concatenate([adj_t[0], pad]).reshape(NW, NROW, SLABW)
    dst = jnp.concatenate([adj_t[1], pad]).reshape(NW, NROW, SLABW)
    xp = jnp.pad(x, ((0, NP - N), (0, 0)))
    zrows = jnp.zeros((NP, D), jnp.float32)
    zdeg = jnp.zeros((NP,), jnp.float32)
    p1, deg0, deg1 = _agg_deg(xp, src, dst, zrows, zdeg)
    d0 = deg0.reshape(NP // D, D)
    d1 = deg1.reshape(NP // D, D)
    h1 = _dense(p1, d0, d1, xp, W1l, b1l.reshape(1, D), W1r, selu=True)
    p2, = _agg(h1, src, dst, zrows)
    out = _dense(p2, d0, d1, h1, W2l, b2l.reshape(1, D), W2r, selu=False)
    return out[:N]
